# Initial kernel scaffold; baseline (speedup 1.0000x reference)
import jax
import jax.numpy as jnp
from jax.experimental import pallas as pl


def kernel(x, edge_index, Wl1, bl1, Wr1, Wl2, bl2, Wr2):
    raise NotImplementedError("write your pallas kernel here")


# speedup vs baseline: 3.0314x; 3.0314x over previous
import jax
import jax.numpy as jnp
from jax import lax
from jax.experimental import pallas as pl
from jax.experimental.pallas import tpu as pltpu
from jax.experimental.pallas import tpu_sc as plsc

N = 10000
E = 320000
D = 128
H = 256

NPAD = 10240
LANES = 128
ER = 2560
EPAD = ER * LANES
NSUB = 16
NCORE = 2
K = 2
RPS = NPAD // NSUB
NCH = RPS // LANES

_f32 = jnp.float32


def _make_deg():
    mesh = plsc.VectorSubcoreMesh(core_axis_name="c", subcore_axis_name="s")
    rpt = ER // (NSUB * NCORE)
    steps = rpt // K

    outs = [jax.ShapeDtypeStruct((NPAD, LANES), _f32),
            jax.ShapeDtypeStruct((NPAD, LANES), _f32)]

    scratch = [
        pltpu.VMEM((K, LANES), jnp.int32),
        pltpu.VMEM((LANES, LANES), _f32),
        pltpu.VMEM_SHARED((NPAD, LANES), _f32),
    ]

    def body(dst2, zd, on, out0, out1, didx, buf, acc):
        c = lax.axis_index("c")
        s = lax.axis_index("s")
        nb = s * RPS

        pltpu.sync_copy(zd, buf)
        for j in range(NCH):
            pltpu.sync_copy(buf, acc.at[pl.ds(nb + j * LANES, LANES)])
        pltpu.sync_copy(on, buf)
        plsc.subcore_barrier()

        base = (c * NSUB + s) * rpt

        def step(i, carry):
            pltpu.sync_copy(dst2.at[pl.ds(base + i * K, K)], didx)
            for k in range(K):
                pltpu.sync_copy(buf.at[pl.ds(0, LANES)],
                                acc.at[didx.at[k]], add=True)
            return carry
        lax.fori_loop(0, steps, step, 0)

        plsc.subcore_barrier()

        def write_out(dst_ref):
            for j in range(NCH):
                pltpu.sync_copy(acc.at[pl.ds(nb + j * LANES, LANES)], buf)
                pltpu.sync_copy(buf,
                                dst_ref.at[pl.ds(nb + j * LANES, LANES)])

        @pl.when(c == 0)
        def _():
            write_out(out0)

        @pl.when(c == 1)
        def _():
            write_out(out1)

    return pl.kernel(body, mesh=mesh, out_type=outs, scratch_types=scratch)


def _make_agg(dh, edge_split):
    mesh = plsc.VectorSubcoreMesh(core_axis_name="c", subcore_axis_name="s")
    nworker = NSUB * NCORE if edge_split else NSUB
    rpt = ER // nworker
    steps = rpt // K

    outs = [jax.ShapeDtypeStruct((NPAD, dh), _f32),
            jax.ShapeDtypeStruct((NPAD, dh), _f32)]

    scratch = [
        pltpu.VMEM((K, LANES), jnp.int32),
        pltpu.VMEM((K, LANES), jnp.int32),
        pltpu.VMEM((K, LANES, dh), _f32),
        pltpu.VMEM_SHARED((NPAD, dh), _f32),
        pltpu.SemaphoreType.DMA,
    ]

    def body(*refs):
        if edge_split:
            (x0, src2, dst2, zd, out0, out1,
             sidx, didx, rows, acc, sem) = refs
            x1 = x0
        else:
            (x0, x1, src2, dst2, zd, out0, out1,
             sidx, didx, rows, acc, sem) = refs

        c = lax.axis_index("c")
        s = lax.axis_index("s")
        nb = s * RPS

        pltpu.sync_copy(zd, rows.at[0])
        for j in range(NCH):
            pltpu.sync_copy(rows.at[0], acc.at[pl.ds(nb + j * LANES, LANES)])
        plsc.subcore_barrier()

        def edge_loop(xref, base):
            def step(i, carry):
                rb = base + i * K
                pltpu.sync_copy(src2.at[pl.ds(rb, K)], sidx)
                pltpu.sync_copy(dst2.at[pl.ds(rb, K)], didx)
                cps = [pltpu.async_copy(xref.at[sidx.at[k]], rows.at[k], sem)
                       for k in range(K)]
                for cp in cps:
                    cp.wait()
                for k in range(K):
                    pltpu.sync_copy(rows.at[k], acc.at[didx.at[k]], add=True)
                return carry
            lax.fori_loop(0, steps, step, 0)

        if edge_split:
            edge_loop(x0, (c * NSUB + s) * rpt)
        else:
            @pl.when(c == 0)
            def _():
                edge_loop(x0, s * rpt)

            @pl.when(c == 1)
            def _():
                edge_loop(x1, s * rpt)

        plsc.subcore_barrier()

        def write_out(dst_ref):
            for j in range(NCH):
                pltpu.sync_copy(acc.at[pl.ds(nb + j * LANES, LANES)],
                                rows.at[0])
                pltpu.sync_copy(rows.at[0],
                                dst_ref.at[pl.ds(nb + j * LANES, LANES)])

        @pl.when(c == 0)
        def _():
            write_out(out0)

        @pl.when(c == 1)
        def _():
            write_out(out1)

    return pl.kernel(body, mesh=mesh, out_type=outs, scratch_types=scratch)


_deg = _make_deg()
_agg1 = _make_agg(D, True)
_agg2 = _make_agg(H // 2, False)


def _tc1_body(p0, p1, d0, d1, xr, wl, wr, b, h0, h1):
    rd = 1.0 / jnp.maximum(d0[:, 0:1] + d1[:, 0:1], 1.0)
    mean = (p0[...] + p1[...]) * rd
    t = (jnp.dot(mean, wl[...], preferred_element_type=_f32)
         + jnp.dot(xr[...], wr[...], preferred_element_type=_f32) + b[...])
    h = jnp.maximum(t, 0.0)
    h0[...] = h[:, :H // 2]
    h1[...] = h[:, H // 2:]


def _tc2_body(a0, a1, d0, d1, h0, h1, wl, wr, b, out):
    rd = 1.0 / jnp.maximum(d0[:, 0:1] + d1[:, 0:1], 1.0)
    mean = jnp.concatenate([a0[...], a1[...]], axis=1) * rd
    hr = jnp.concatenate([h0[...], h1[...]], axis=1)
    out[...] = (jnp.dot(mean, wl[...], preferred_element_type=_f32)
                + jnp.dot(hr[...], wr[...], preferred_element_type=_f32)
                + b[...])


BT = 1024


def _row_spec(cols):
    return pl.BlockSpec((BT, cols), lambda i: (i, 0))


def _full_spec(r, cols):
    return pl.BlockSpec((r, cols), lambda i: (0, 0))


_tc1 = pl.pallas_call(
    _tc1_body,
    grid=(NPAD // BT,),
    in_specs=[_row_spec(D), _row_spec(D), _row_spec(LANES), _row_spec(LANES),
              _row_spec(D), _full_spec(D, H), _full_spec(D, H),
              _full_spec(1, H)],
    out_specs=[_row_spec(H // 2), _row_spec(H // 2)],
    out_shape=[jax.ShapeDtypeStruct((NPAD, H // 2), _f32)] * 2,
)

_tc2 = pl.pallas_call(
    _tc2_body,
    grid=(NPAD // BT,),
    in_specs=[_row_spec(H // 2), _row_spec(H // 2),
              _row_spec(LANES), _row_spec(LANES),
              _row_spec(H // 2), _row_spec(H // 2),
              _full_spec(H, H), _full_spec(H, H), _full_spec(1, H)],
    out_specs=_row_spec(H),
    out_shape=jax.ShapeDtypeStruct((NPAD, H), _f32),
)


def kernel(x, edge_index, Wl1, bl1, Wr1, Wl2, bl2, Wr2):
    src = edge_index[0]
    dst = edge_index[1]
    src_p = jnp.concatenate(
        [src, jnp.zeros((EPAD - E,), jnp.int32)]).reshape(ER, LANES)
    dst_p = jnp.concatenate(
        [dst, jnp.full((EPAD - E,), N, jnp.int32)]).reshape(ER, LANES)
    xp = jnp.pad(x, ((0, NPAD - N), (0, 0)))

    zd1 = jnp.zeros((LANES, D), _f32)
    zd2 = jnp.zeros((LANES, H // 2), _f32)
    ones2d = jnp.ones((LANES, LANES), _f32)

    d0, d1 = _deg(dst_p, zd1, ones2d)
    p0, p1 = _agg1(xp, src_p, dst_p, zd1)
    h0, h1 = _tc1(p0, p1, d0, d1, xp, Wl1.T, Wr1.T, bl1.reshape(1, H))
    b0, b1 = _agg2(h0, h1, src_p, dst_p, zd2)
    out = _tc2(b0, b1, d0, d1, h0, h1, Wl2.T, Wr2.T, bl2.reshape(1, H))
    return out[:N]


# speedup vs baseline: 3.5425x; 1.1686x over previous
import jax
import jax.numpy as jnp
from jax import lax
from jax.experimental import pallas as pl
from jax.experimental.pallas import tpu as pltpu
from jax.experimental.pallas import tpu_sc as plsc

N = 10000
E = 320000
D = 128
H = 256

NPAD = 10240
LANES = 128
ER = 2560
EPAD = ER * LANES
NSUB = 16
NCORE = 2
K = 2
RPS = NPAD // NSUB
NCH = RPS // LANES

_f32 = jnp.float32


def _make_deg():
    mesh = plsc.VectorSubcoreMesh(core_axis_name="c", subcore_axis_name="s")
    rpt = ER // (NSUB * NCORE)
    steps = rpt // K

    outs = [jax.ShapeDtypeStruct((NPAD, LANES), _f32),
            jax.ShapeDtypeStruct((NPAD, LANES), _f32)]

    scratch = [
        pltpu.VMEM((K, LANES), jnp.int32),
        pltpu.VMEM((LANES, LANES), _f32),
        pltpu.VMEM_SHARED((NPAD, LANES), _f32),
    ]

    def body(dst2, zd, on, out0, out1, didx, buf, acc):
        c = lax.axis_index("c")
        s = lax.axis_index("s")
        nb = s * RPS

        pltpu.sync_copy(zd, buf)
        for j in range(NCH):
            pltpu.sync_copy(buf, acc.at[pl.ds(nb + j * LANES, LANES)])
        pltpu.sync_copy(on, buf)
        plsc.subcore_barrier()

        base = (c * NSUB + s) * rpt

        def step(i, carry):
            pltpu.sync_copy(dst2.at[pl.ds(base + i * K, K)], didx)
            for k in range(K):
                pltpu.sync_copy(buf.at[pl.ds(0, LANES)],
                                acc.at[didx.at[k]], add=True)
            return carry
        lax.fori_loop(0, steps, step, 0)

        plsc.subcore_barrier()

        def write_out(dst_ref):
            for j in range(NCH):
                pltpu.sync_copy(acc.at[pl.ds(nb + j * LANES, LANES)], buf)
                pltpu.sync_copy(buf,
                                dst_ref.at[pl.ds(nb + j * LANES, LANES)])

        @pl.when(c == 0)
        def _():
            write_out(out0)

        @pl.when(c == 1)
        def _():
            write_out(out1)

    return pl.kernel(body, mesh=mesh, out_type=outs, scratch_types=scratch)


def _make_agg(dh, edge_split):
    mesh = plsc.VectorSubcoreMesh(core_axis_name="c", subcore_axis_name="s")
    nworker = NSUB * NCORE if edge_split else NSUB
    rpt = ER // nworker
    IB = 16
    nbatch = rpt // IB

    outs = [jax.ShapeDtypeStruct((NPAD, dh), _f32),
            jax.ShapeDtypeStruct((NPAD, dh), _f32)]

    scratch = [
        pltpu.VMEM((IB, LANES), jnp.int32),
        pltpu.VMEM((IB, LANES), jnp.int32),
        pltpu.VMEM((2, LANES, dh), _f32),
        pltpu.VMEM_SHARED((NPAD, dh), _f32),
        pltpu.SemaphoreType.DMA,
        pltpu.SemaphoreType.DMA,
    ]

    def body(*refs):
        if edge_split:
            (x0, src2, dst2, zd, out0, out1,
             sidx, didx, rows, acc, sem0, sem1) = refs
            x1 = x0
        else:
            (x0, x1, src2, dst2, zd, out0, out1,
             sidx, didx, rows, acc, sem0, sem1) = refs

        sems = (sem0, sem1)
        c = lax.axis_index("c")
        s = lax.axis_index("s")
        nb = s * RPS

        pltpu.sync_copy(zd, rows.at[0])
        for j in range(NCH):
            pltpu.sync_copy(rows.at[0], acc.at[pl.ds(nb + j * LANES, LANES)])
        plsc.subcore_barrier()

        def edge_loop(xref, base):
            def step(i, carry):
                pltpu.sync_copy(src2.at[pl.ds(base + i * IB, IB)], sidx)
                pltpu.sync_copy(dst2.at[pl.ds(base + i * IB, IB)], didx)
                cps = [None, None]
                cps[0] = pltpu.async_copy(
                    xref.at[sidx.at[0]], rows.at[0], sems[0])
                for j in range(IB):
                    sl = j % 2
                    if j + 1 < IB:
                        cps[1 - sl] = pltpu.async_copy(
                            xref.at[sidx.at[j + 1]], rows.at[1 - sl],
                            sems[1 - sl])
                    cps[sl].wait()
                    pltpu.sync_copy(rows.at[sl], acc.at[didx.at[j]],
                                    add=True)
                return carry
            lax.fori_loop(0, nbatch, step, 0)

        if edge_split:
            edge_loop(x0, (c * NSUB + s) * rpt)
        else:
            @pl.when(c == 0)
            def _():
                edge_loop(x0, s * rpt)

            @pl.when(c == 1)
            def _():
                edge_loop(x1, s * rpt)

        plsc.subcore_barrier()

        def write_out(dst_ref):
            for j in range(NCH):
                pltpu.sync_copy(acc.at[pl.ds(nb + j * LANES, LANES)],
                                rows.at[0])
                pltpu.sync_copy(rows.at[0],
                                dst_ref.at[pl.ds(nb + j * LANES, LANES)])

        @pl.when(c == 0)
        def _():
            write_out(out0)

        @pl.when(c == 1)
        def _():
            write_out(out1)

    return pl.kernel(body, mesh=mesh, out_type=outs, scratch_types=scratch)


_deg = _make_deg()
_agg1 = _make_agg(D, True)
_agg2 = _make_agg(H // 2, False)


def _tc1_body(p0, p1, d0, d1, xr, wl, wr, b, h0, h1):
    rd = 1.0 / jnp.maximum(d0[:, 0:1] + d1[:, 0:1], 1.0)
    mean = (p0[...] + p1[...]) * rd
    t = (jnp.dot(mean, wl[...], preferred_element_type=_f32)
         + jnp.dot(xr[...], wr[...], preferred_element_type=_f32) + b[...])
    h = jnp.maximum(t, 0.0)
    h0[...] = h[:, :H // 2]
    h1[...] = h[:, H // 2:]


def _tc2_body(a0, a1, d0, d1, h0, h1, wl, wr, b, out):
    rd = 1.0 / jnp.maximum(d0[:, 0:1] + d1[:, 0:1], 1.0)
    mean = jnp.concatenate([a0[...], a1[...]], axis=1) * rd
    hr = jnp.concatenate([h0[...], h1[...]], axis=1)
    out[...] = (jnp.dot(mean, wl[...], preferred_element_type=_f32)
                + jnp.dot(hr[...], wr[...], preferred_element_type=_f32)
                + b[...])


BT = 1024


def _row_spec(cols):
    return pl.BlockSpec((BT, cols), lambda i: (i, 0))


def _full_spec(r, cols):
    return pl.BlockSpec((r, cols), lambda i: (0, 0))


_tc1 = pl.pallas_call(
    _tc1_body,
    grid=(NPAD // BT,),
    in_specs=[_row_spec(D), _row_spec(D), _row_spec(LANES), _row_spec(LANES),
              _row_spec(D), _full_spec(D, H), _full_spec(D, H),
              _full_spec(1, H)],
    out_specs=[_row_spec(H // 2), _row_spec(H // 2)],
    out_shape=[jax.ShapeDtypeStruct((NPAD, H // 2), _f32)] * 2,
)

_tc2 = pl.pallas_call(
    _tc2_body,
    grid=(NPAD // BT,),
    in_specs=[_row_spec(H // 2), _row_spec(H // 2),
              _row_spec(LANES), _row_spec(LANES),
              _row_spec(H // 2), _row_spec(H // 2),
              _full_spec(H, H), _full_spec(H, H), _full_spec(1, H)],
    out_specs=_row_spec(H),
    out_shape=jax.ShapeDtypeStruct((NPAD, H), _f32),
)


def kernel(x, edge_index, Wl1, bl1, Wr1, Wl2, bl2, Wr2):
    src = edge_index[0]
    dst = edge_index[1]
    src_p = jnp.concatenate(
        [src, jnp.zeros((EPAD - E,), jnp.int32)]).reshape(ER, LANES)
    dst_p = jnp.concatenate(
        [dst, jnp.full((EPAD - E,), N, jnp.int32)]).reshape(ER, LANES)
    xp = jnp.pad(x, ((0, NPAD - N), (0, 0)))

    zd1 = jnp.zeros((LANES, D), _f32)
    zd2 = jnp.zeros((LANES, H // 2), _f32)
    ones2d = jnp.ones((LANES, LANES), _f32)

    d0, d1 = _deg(dst_p, zd1, ones2d)
    p0, p1 = _agg1(xp, src_p, dst_p, zd1)
    h0, h1 = _tc1(p0, p1, d0, d1, xp, Wl1.T, Wr1.T, bl1.reshape(1, H))
    b0, b1 = _agg2(h0, h1, src_p, dst_p, zd2)
    out = _tc2(b0, b1, d0, d1, h0, h1, Wl2.T, Wr2.T, bl2.reshape(1, H))
    return out[:N]


# speedup vs baseline: 3.8685x; 1.0920x over previous
import jax
import jax.numpy as jnp
from jax import lax
from jax.experimental import pallas as pl
from jax.experimental.pallas import tpu as pltpu
from jax.experimental.pallas import tpu_sc as plsc

N = 10000
E = 320000
D = 128
H = 256

NPAD = 10240
LANES = 128
ER = 2560
EPAD = ER * LANES
NSUB = 16
NCORE = 2
K = 2
RPS = NPAD // NSUB
NCH = RPS // LANES

_f32 = jnp.float32


def _make_deg():
    mesh = plsc.VectorSubcoreMesh(core_axis_name="c", subcore_axis_name="s")
    rpt = ER // (NSUB * NCORE)
    steps = rpt // K

    outs = [jax.ShapeDtypeStruct((NPAD, LANES), _f32),
            jax.ShapeDtypeStruct((NPAD, LANES), _f32)]

    scratch = [
        pltpu.VMEM((K, LANES), jnp.int32),
        pltpu.VMEM((LANES, LANES), _f32),
        pltpu.VMEM_SHARED((NPAD, LANES), _f32),
    ]

    def body(dst2, zd, on, out0, out1, didx, buf, acc):
        c = lax.axis_index("c")
        s = lax.axis_index("s")
        nb = s * RPS

        pltpu.sync_copy(zd, buf)
        for j in range(NCH):
            pltpu.sync_copy(buf, acc.at[pl.ds(nb + j * LANES, LANES)])
        pltpu.sync_copy(on, buf)
        plsc.subcore_barrier()

        base = (c * NSUB + s) * rpt

        def step(i, carry):
            pltpu.sync_copy(dst2.at[pl.ds(base + i * K, K)], didx)
            for k in range(K):
                pltpu.sync_copy(buf.at[pl.ds(0, LANES)],
                                acc.at[didx.at[k]], add=True)
            return carry
        lax.fori_loop(0, steps, step, 0)

        plsc.subcore_barrier()

        def write_out(dst_ref):
            for j in range(NCH):
                pltpu.sync_copy(acc.at[pl.ds(nb + j * LANES, LANES)], buf)
                pltpu.sync_copy(buf,
                                dst_ref.at[pl.ds(nb + j * LANES, LANES)])

        @pl.when(c == 0)
        def _():
            write_out(out0)

        @pl.when(c == 1)
        def _():
            write_out(out1)

    return pl.kernel(body, mesh=mesh, out_type=outs, scratch_types=scratch)


def _make_agg(dh, edge_split):
    mesh = plsc.VectorSubcoreMesh(core_axis_name="c", subcore_axis_name="s")
    nworker = NSUB * NCORE if edge_split else NSUB
    rpt = ER // nworker
    IB = 32
    nbatch = rpt // IB

    outs = [jax.ShapeDtypeStruct((NPAD, dh), _f32),
            jax.ShapeDtypeStruct((NPAD, dh), _f32)]

    scratch = [
        pltpu.VMEM((IB, LANES), jnp.int32),
        pltpu.VMEM((IB, LANES), jnp.int32),
        pltpu.VMEM((2, LANES, dh), _f32),
        pltpu.VMEM_SHARED((NPAD, dh), _f32),
        pltpu.SemaphoreType.DMA,
        pltpu.SemaphoreType.DMA,
    ]

    def body(*refs):
        if edge_split:
            (x0, src2, dst2, zd, out0, out1,
             sidx, didx, rows, acc, sem0, sem1) = refs
            x1 = x0
        else:
            (x0, x1, src2, dst2, zd, out0, out1,
             sidx, didx, rows, acc, sem0, sem1) = refs

        sems = (sem0, sem1)
        c = lax.axis_index("c")
        s = lax.axis_index("s")
        nb = s * RPS

        pltpu.sync_copy(zd, rows.at[0])
        for j in range(NCH):
            pltpu.sync_copy(rows.at[0], acc.at[pl.ds(nb + j * LANES, LANES)])
        plsc.subcore_barrier()

        def edge_loop(xref, base):
            def step(i, carry):
                pltpu.sync_copy(src2.at[pl.ds(base + i * IB, IB)], sidx)
                pltpu.sync_copy(dst2.at[pl.ds(base + i * IB, IB)], didx)
                cps = [None, None]
                cps[0] = pltpu.async_copy(
                    xref.at[sidx.at[0]], rows.at[0], sems[0])
                for j in range(IB):
                    sl = j % 2
                    if j + 1 < IB:
                        cps[1 - sl] = pltpu.async_copy(
                            xref.at[sidx.at[j + 1]], rows.at[1 - sl],
                            sems[1 - sl])
                    cps[sl].wait()
                    pltpu.sync_copy(rows.at[sl], acc.at[didx.at[j]],
                                    add=True)
                return carry
            lax.fori_loop(0, nbatch, step, 0)

        if edge_split:
            edge_loop(x0, (c * NSUB + s) * rpt)
        else:
            @pl.when(c == 0)
            def _():
                edge_loop(x0, s * rpt)

            @pl.when(c == 1)
            def _():
                edge_loop(x1, s * rpt)

        plsc.subcore_barrier()

        def write_out(dst_ref):
            for j in range(NCH):
                pltpu.sync_copy(acc.at[pl.ds(nb + j * LANES, LANES)],
                                rows.at[0])
                pltpu.sync_copy(rows.at[0],
                                dst_ref.at[pl.ds(nb + j * LANES, LANES)])

        @pl.when(c == 0)
        def _():
            write_out(out0)

        @pl.when(c == 1)
        def _():
            write_out(out1)

    return pl.kernel(body, mesh=mesh, out_type=outs, scratch_types=scratch)


_deg = _make_deg()
_agg1 = _make_agg(D, True)
_agg2 = _make_agg(H // 2, False)


def _tc1_body(p0, p1, d0, d1, xr, wl, wr, b, h0, h1):
    rd = 1.0 / jnp.maximum(d0[:, 0:1] + d1[:, 0:1], 1.0)
    mean = (p0[...] + p1[...]) * rd
    t = (jnp.dot(mean, wl[...], preferred_element_type=_f32)
         + jnp.dot(xr[...], wr[...], preferred_element_type=_f32) + b[...])
    h = jnp.maximum(t, 0.0)
    h0[...] = h[:, :H // 2]
    h1[...] = h[:, H // 2:]


def _tc2_body(a0, a1, d0, d1, h0, h1, wl, wr, b, out):
    rd = 1.0 / jnp.maximum(d0[:, 0:1] + d1[:, 0:1], 1.0)
    mean = jnp.concatenate([a0[...], a1[...]], axis=1) * rd
    hr = jnp.concatenate([h0[...], h1[...]], axis=1)
    out[...] = (jnp.dot(mean, wl[...], preferred_element_type=_f32)
                + jnp.dot(hr[...], wr[...], preferred_element_type=_f32)
                + b[...])


BT = 1024


def _row_spec(cols):
    return pl.BlockSpec((BT, cols), lambda i: (i, 0))


def _full_spec(r, cols):
    return pl.BlockSpec((r, cols), lambda i: (0, 0))


_tc1 = pl.pallas_call(
    _tc1_body,
    grid=(NPAD // BT,),
    in_specs=[_row_spec(D), _row_spec(D), _row_spec(LANES), _row_spec(LANES),
              _row_spec(D), _full_spec(D, H), _full_spec(D, H),
              _full_spec(1, H)],
    out_specs=[_row_spec(H // 2), _row_spec(H // 2)],
    out_shape=[jax.ShapeDtypeStruct((NPAD, H // 2), _f32)] * 2,
)

_tc2 = pl.pallas_call(
    _tc2_body,
    grid=(NPAD // BT,),
    in_specs=[_row_spec(H // 2), _row_spec(H // 2),
              _row_spec(LANES), _row_spec(LANES),
              _row_spec(H // 2), _row_spec(H // 2),
              _full_spec(H, H), _full_spec(H, H), _full_spec(1, H)],
    out_specs=_row_spec(H),
    out_shape=jax.ShapeDtypeStruct((NPAD, H), _f32),
)


def kernel(x, edge_index, Wl1, bl1, Wr1, Wl2, bl2, Wr2):
    src = edge_index[0]
    dst = edge_index[1]
    src_p = jnp.concatenate(
        [src, jnp.zeros((EPAD - E,), jnp.int32)]).reshape(ER, LANES)
    pad_dst = N + jnp.arange(EPAD - E, dtype=jnp.int32) % (NPAD - N)
    dst_p = jnp.concatenate([dst, pad_dst]).reshape(ER, LANES)
    xp = jnp.pad(x, ((0, NPAD - N), (0, 0)))

    zd1 = jnp.zeros((LANES, D), _f32)
    zd2 = jnp.zeros((LANES, H // 2), _f32)
    ones2d = jnp.ones((LANES, LANES), _f32)

    d0, d1 = _deg(dst_p, zd1, ones2d)
    p0, p1 = _agg1(xp, src_p, dst_p, zd1)
    h0, h1 = _tc1(p0, p1, d0, d1, xp, Wl1.T, Wr1.T, bl1.reshape(1, H))
    b0, b1 = _agg2(h0, h1, src_p, dst_p, zd2)
    out = _tc2(b0, b1, d0, d1, h0, h1, Wl2.T, Wr2.T, bl2.reshape(1, H))
    return out[:N]
